# baseline (speedup 1.0000x reference)
import jax
import jax.numpy as jnp
from jax.experimental import pallas as pl
from jax.experimental.pallas import tpu as pltpu

_B = 4096
_D = 2048
_CPAD = 128
_BB = 1024
_G = _B // _BB


def _body(labels_ref, f_ref, out_ref, acc_ref):
    i = pl.program_id(0)
    f = f_ref[...]
    fb = f.astype(jnp.bfloat16)
    ones = jnp.ones((_D, _CPAD), jnp.bfloat16)
    sumsq = jax.lax.dot_general(
        fb * fb, ones, (((1,), (0,)), ((), ())),
        preferred_element_type=jnp.float32,
        precision=jax.lax.Precision.DEFAULT)
    inv = jax.lax.rsqrt(jnp.maximum(sumsq, 1e-24))
    lab = labels_ref[0, 0, :]
    cls = jax.lax.broadcasted_iota(jnp.int32, (_BB, _CPAD), 1)
    oh = jnp.where(lab[:, None] == cls, inv, 0.0)
    part = jax.lax.dot_general(
        oh.astype(jnp.bfloat16), fb, (((0,), (0,)), ((), ())),
        preferred_element_type=jnp.float32,
        precision=jax.lax.Precision.DEFAULT)

    @pl.when(i == 0)
    def _():
        acc_ref[...] = part

    @pl.when(i > 0)
    def _():
        acc_ref[...] += part

    @pl.when(i == _G - 1)
    def _():
        s = acc_ref[...]
        normsq = jnp.sum(s * s, axis=1)
        total = jnp.sum(jnp.sqrt(normsq))
        out_ref[...] = jnp.full((1, 1), 1.0, jnp.float32) - total / _B


def kernel(features, labels):
    labels3 = labels.astype(jnp.int32).reshape(_G, 1, _BB)
    out = pl.pallas_call(
        _body,
        grid=(_G,),
        in_specs=[
            pl.BlockSpec((1, 1, _BB), lambda i: (i, 0, 0)),
            pl.BlockSpec((_BB, _D), lambda i: (i, 0)),
        ],
        out_specs=pl.BlockSpec((1, 1), lambda i: (0, 0)),
        out_shape=jax.ShapeDtypeStruct((1, 1), jnp.float32),
        scratch_shapes=[pltpu.VMEM((_CPAD, _D), jnp.float32)],
    )(labels3, features)
    return out[0, 0]


# speedup vs baseline: 1.1520x; 1.1520x over previous
import jax
import jax.numpy as jnp
from jax.experimental import pallas as pl
from jax.experimental.pallas import tpu as pltpu

_B = 4096
_D = 2048
_CPAD = 128
_BB = 512
_G = _B // _BB


def _body(labels_ref, f_ref, out_ref, acc_ref):
    i = pl.program_id(0)
    f = f_ref[...]
    sumsq = jnp.sum(f * f, axis=1)
    inv = jax.lax.rsqrt(jnp.maximum(sumsq, 1e-24))
    lab = labels_ref[0, 0, :]
    cls = jax.lax.broadcasted_iota(jnp.int32, (_BB, _CPAD), 1)
    oh = jnp.where(lab[:, None] == cls, inv[:, None], 0.0)
    part = jax.lax.dot_general(
        oh.astype(jnp.bfloat16), f.astype(jnp.bfloat16), (((0,), (0,)), ((), ())),
        preferred_element_type=jnp.float32,
        precision=jax.lax.Precision.DEFAULT)

    @pl.when(i == 0)
    def _():
        acc_ref[...] = part

    @pl.when(i > 0)
    def _():
        acc_ref[...] += part

    @pl.when(i == _G - 1)
    def _():
        s = acc_ref[...]
        normsq = jnp.sum(s * s, axis=1)
        total = jnp.sum(jnp.sqrt(normsq))
        out_ref[...] = jnp.full((1, 1), 1.0, jnp.float32) - total / _B


def kernel(features, labels):
    labels3 = labels.astype(jnp.int32).reshape(_G, 1, _BB)
    out = pl.pallas_call(
        _body,
        grid=(_G,),
        in_specs=[
            pl.BlockSpec((1, 1, _BB), lambda i: (i, 0, 0)),
            pl.BlockSpec((_BB, _D), lambda i: (i, 0)),
        ],
        out_specs=pl.BlockSpec((1, 1), lambda i: (0, 0)),
        out_shape=jax.ShapeDtypeStruct((1, 1), jnp.float32),
        scratch_shapes=[pltpu.VMEM((_CPAD, _D), jnp.float32)],
    )(labels3, features)
    return out[0, 0]


# speedup vs baseline: 1.3152x; 1.1416x over previous
import jax
import jax.numpy as jnp
from jax.experimental import pallas as pl
from jax.experimental.pallas import tpu as pltpu

_B = 4096
_D = 2048
_CPAD = 128
_BB = 1024
_G = _B // _BB


def _body(labels_ref, f_ref, out_ref, acc_ref):
    i = pl.program_id(0)
    f = f_ref[...]
    p = None
    for j in range(_D // 128):
        sl = f[:, j * 128:(j + 1) * 128]
        p = sl * sl if p is None else p + sl * sl
    ones = jnp.ones((128, _CPAD), jnp.float32)
    sumsq = jax.lax.dot_general(
        p, ones, (((1,), (0,)), ((), ())),
        preferred_element_type=jnp.float32,
        precision=jax.lax.Precision.DEFAULT)
    inv = jax.lax.rsqrt(jnp.maximum(sumsq, 1e-24))
    lab = labels_ref[0, 0, :]
    cls = jax.lax.broadcasted_iota(jnp.int32, (_BB, _CPAD), 1)
    oh = jnp.where(lab[:, None] == cls, inv, 0.0)
    part = jax.lax.dot_general(
        oh.astype(jnp.bfloat16), f.astype(jnp.bfloat16), (((0,), (0,)), ((), ())),
        preferred_element_type=jnp.float32,
        precision=jax.lax.Precision.DEFAULT)

    @pl.when(i == 0)
    def _():
        acc_ref[...] = part

    @pl.when(i > 0)
    def _():
        acc_ref[...] += part

    @pl.when(i == _G - 1)
    def _():
        s = acc_ref[...]
        normsq = jnp.sum(s * s, axis=1)
        total = jnp.sum(jnp.sqrt(normsq))
        out_ref[...] = jnp.full((1, 1), 1.0, jnp.float32) - total / _B


def kernel(features, labels):
    labels3 = labels.astype(jnp.int32).reshape(_G, 1, _BB)
    out = pl.pallas_call(
        _body,
        grid=(_G,),
        in_specs=[
            pl.BlockSpec((1, 1, _BB), lambda i: (i, 0, 0)),
            pl.BlockSpec((_BB, _D), lambda i: (i, 0)),
        ],
        out_specs=pl.BlockSpec((1, 1), lambda i: (0, 0)),
        out_shape=jax.ShapeDtypeStruct((1, 1), jnp.float32),
        scratch_shapes=[pltpu.VMEM((_CPAD, _D), jnp.float32)],
    )(labels3, features)
    return out[0, 0]
